# baseline (speedup 1.0000x reference)
import functools

import jax
import jax.numpy as jnp
from jax import lax
from jax.experimental import pallas as pl
from jax.experimental.pallas import tpu as pltpu
from jax.experimental.pallas import tpu_sc as plsc

CHUNK = 1280


@functools.cache
def _build(B: int, V: int, D: int):
    info = plsc.get_sparse_core_info()
    NW = info.num_cores * info.num_subcores
    b_per_w = B // NW
    n_chunks = b_per_w // CHUNK
    assert n_chunks % 2 == 0 and n_chunks >= 4
    mesh = plsc.VectorSubcoreMesh(core_axis_name="c", subcore_axis_name="s")

    scratch = (
        [pltpu.VMEM((CHUNK,), jnp.int32) for _ in range(2)]
        + [pltpu.VMEM((CHUNK, D), jnp.float32) for _ in range(2)]
        + [pltpu.SemaphoreType.DMA for _ in range(6)]
    )

    @functools.partial(
        pl.kernel,
        mesh=mesh,
        out_type=jax.ShapeDtypeStruct((B, D), jnp.float32),
        scratch_types=scratch,
        compiler_params=pltpu.CompilerParams(use_tc_tiling_on_sc=False),
    )
    def gather_kernel(idx_hbm, table_hbm, out_hbm, iv0, iv1, rv0, rv1,
                      si0, si1, sg0, sg1, so0, so1):
        iv = (iv0, iv1)
        rv = (rv0, rv1)
        si = (si0, si1)
        sg = (sg0, sg1)
        so = (so0, so1)
        wid = lax.axis_index("s") * info.num_cores + lax.axis_index("c")
        base = wid * b_per_w

        def idx_slice(i):
            return idx_hbm.at[pl.ds(base + i * CHUNK, CHUNK)]

        def out_slice(i):
            return out_hbm.at[pl.ds(base + i * CHUNK, CHUNK)]

        pltpu.async_copy(idx_slice(0), iv0, si0)
        pltpu.async_copy(idx_slice(1), iv1, si1)
        pltpu.make_async_copy(idx_slice(0), iv0, si0).wait()
        pltpu.async_copy(table_hbm.at[iv0], rv0, sg0)

        def body(g, carry):
            for b in range(2):
                o = 1 - b
                i = 2 * g + b
                pltpu.make_async_copy(table_hbm.at[iv[b]], rv[b], sg[b]).wait()
                pltpu.async_copy(rv[b], out_slice(i), so[b])
                pltpu.async_copy(idx_slice(lax.rem(i + 2, n_chunks)),
                                 iv[b], si[b])

                @pl.when(i < n_chunks - 1)
                def _():
                    pltpu.make_async_copy(idx_slice(i + 1), iv[o], si[o]).wait()

                    @pl.when(i >= 1)
                    def _():
                        pltpu.make_async_copy(
                            rv[o], out_slice(i - 1), so[o]).wait()

                    pltpu.async_copy(table_hbm.at[iv[o]], rv[o], sg[o])
            return carry

        lax.fori_loop(0, n_chunks // 2, body, 0)
        pltpu.make_async_copy(rv0, out_slice(n_chunks - 2), so0).wait()
        pltpu.make_async_copy(rv1, out_slice(n_chunks - 1), so1).wait()
        pltpu.make_async_copy(idx_slice(0), iv0, si0).wait()
        pltpu.make_async_copy(idx_slice(1), iv1, si1).wait()

    return gather_kernel


def kernel(radius, table):
    B0, H = radius.shape
    V, D = table.shape
    flat_idx = radius.reshape(-1).astype(jnp.int32)
    out = _build(B0 * H, V, D)(flat_idx, table)
    return out.reshape(B0, H, D)


# speedup vs baseline: 1.0019x; 1.0019x over previous
import functools

import jax
import jax.numpy as jnp
from jax import lax
from jax.experimental import pallas as pl
from jax.experimental.pallas import tpu as pltpu
from jax.experimental.pallas import tpu_sc as plsc

CHUNK = 1600


@functools.cache
def _build(B: int, V: int, D: int):
    info = plsc.get_sparse_core_info()
    NW = info.num_cores * info.num_subcores
    b_per_w = B // NW
    n_chunks = b_per_w // CHUNK
    assert n_chunks % 2 == 0 and n_chunks >= 4
    mesh = plsc.VectorSubcoreMesh(core_axis_name="c", subcore_axis_name="s")

    scratch = (
        [pltpu.VMEM((CHUNK,), jnp.int32) for _ in range(2)]
        + [pltpu.VMEM((CHUNK, D), jnp.float32) for _ in range(2)]
        + [pltpu.SemaphoreType.DMA for _ in range(6)]
    )

    @functools.partial(
        pl.kernel,
        mesh=mesh,
        out_type=jax.ShapeDtypeStruct((B, D), jnp.float32),
        scratch_types=scratch,
        compiler_params=pltpu.CompilerParams(use_tc_tiling_on_sc=False),
    )
    def gather_kernel(idx_hbm, table_hbm, out_hbm, iv0, iv1, rv0, rv1,
                      si0, si1, sg0, sg1, so0, so1):
        iv = (iv0, iv1)
        rv = (rv0, rv1)
        si = (si0, si1)
        sg = (sg0, sg1)
        so = (so0, so1)
        wid = lax.axis_index("s") * info.num_cores + lax.axis_index("c")
        base = wid * b_per_w

        def idx_slice(i):
            return idx_hbm.at[pl.ds(base + i * CHUNK, CHUNK)]

        def out_slice(i):
            return out_hbm.at[pl.ds(base + i * CHUNK, CHUNK)]

        pltpu.async_copy(idx_slice(0), iv0, si0)
        pltpu.async_copy(idx_slice(1), iv1, si1)
        pltpu.make_async_copy(idx_slice(0), iv0, si0).wait()
        pltpu.async_copy(table_hbm.at[iv0], rv0, sg0)

        def body(g, carry):
            for b in range(2):
                o = 1 - b
                i = 2 * g + b
                pltpu.make_async_copy(table_hbm.at[iv[b]], rv[b], sg[b]).wait()
                pltpu.async_copy(rv[b], out_slice(i), so[b])
                pltpu.async_copy(idx_slice(lax.rem(i + 2, n_chunks)),
                                 iv[b], si[b])

                @pl.when(i < n_chunks - 1)
                def _():
                    pltpu.make_async_copy(idx_slice(i + 1), iv[o], si[o]).wait()

                    @pl.when(i >= 1)
                    def _():
                        pltpu.make_async_copy(
                            rv[o], out_slice(i - 1), so[o]).wait()

                    pltpu.async_copy(table_hbm.at[iv[o]], rv[o], sg[o])
            return carry

        lax.fori_loop(0, n_chunks // 2, body, 0)
        pltpu.make_async_copy(rv0, out_slice(n_chunks - 2), so0).wait()
        pltpu.make_async_copy(rv1, out_slice(n_chunks - 1), so1).wait()
        pltpu.make_async_copy(idx_slice(0), iv0, si0).wait()
        pltpu.make_async_copy(idx_slice(1), iv1, si1).wait()

    return gather_kernel


def kernel(radius, table):
    B0, H = radius.shape
    V, D = table.shape
    flat_idx = radius.reshape(-1).astype(jnp.int32)
    out = _build(B0 * H, V, D)(flat_idx, table)
    return out.reshape(B0, H, D)
